# baseline (speedup 1.0000x reference)
import functools
import math

import jax
import jax.numpy as jnp
from jax import lax
from jax.experimental import pallas as pl
from jax.experimental.pallas import tpu as pltpu
from jax.experimental.pallas import tpu_sc as plsc

VOCAB = 100000
D = 128
B = 4096
H = 20
HPAD = 24
NC, NS = 2, 16
NW = NC * NS
FLAT = B * HPAD
PER_W = FLAT // NW
CHUNK = 128
NCH = PER_W // CHUNK
NBUF = 3
SCALE = float(math.sqrt(float(D)))

_mesh = plsc.VectorSubcoreMesh(core_axis_name="c", subcore_axis_name="s")


@functools.partial(
    pl.kernel,
    out_type=jax.ShapeDtypeStruct((FLAT, D), jnp.float32),
    mesh=_mesh,
    scratch_types=[
        pltpu.VMEM((NCH, CHUNK), jnp.int32),
        pltpu.VMEM((CHUNK, D), jnp.float32),
        pltpu.VMEM((CHUNK, D), jnp.float32),
        pltpu.SemaphoreType.DMA,
        pltpu.SemaphoreType.DMA,
    ],
    compiler_params=pltpu.CompilerParams(use_tc_tiling_on_sc=True),
)
def _embed_gather(idx_hbm, table_hbm, out_hbm, idx_v, buf_a, buf_b, sem_a, sem_b):
    bufs = (buf_a, buf_b)
    sems = (sem_a, sem_b)
    wid = lax.axis_index("s") * NC + lax.axis_index("c")
    base = wid * PER_W

    pltpu.sync_copy(idx_hbm.at[pl.ds(wid * NCH, NCH)], idx_v)

    pltpu.async_copy(table_hbm.at[idx_v.at[0]], bufs[0], sems[0])

    for j in range(NCH):
        buf = bufs[j % 2]
        pltpu.make_async_copy(table_hbm.at[idx_v.at[j]], buf, sems[j % 2]).wait()
        if j + 1 < NCH:
            pltpu.async_copy(
                table_hbm.at[idx_v.at[j + 1]], bufs[(j + 1) % 2], sems[(j + 1) % 2]
            )

        def scale_row(r, _, buf=buf):
            for q in range(D // 16):
                buf[r, pl.ds(q * 16, 16)] = buf[r, pl.ds(q * 16, 16)] * SCALE
            return 0

        lax.fori_loop(0, CHUNK, scale_row, 0)
        pltpu.sync_copy(buf, out_hbm.at[pl.ds(base + j * CHUNK, CHUNK)])


def kernel(x, input_embedding_table):
    idx = jnp.pad(x.astype(jnp.int32), ((0, 0), (0, HPAD - H)))
    idx = idx.reshape(NW * NCH, CHUNK)
    framed = _embed_gather(idx, input_embedding_table)
    return framed.reshape(B, HPAD, D)[:, :H, :]


# speedup vs baseline: 7.9545x; 7.9545x over previous
import functools
import math

import jax
import jax.numpy as jnp
from jax import lax
from jax.experimental import pallas as pl
from jax.experimental.pallas import tpu as pltpu
from jax.experimental.pallas import tpu_sc as plsc

VOCAB = 100000
D = 128
B = 4096
H = 20
HPAD = 24
NC, NS = 2, 16
NW = NC * NS
FLAT = B * HPAD
PER_W = FLAT // NW
CHUNK = 128
NCH = PER_W // CHUNK
NBUF = 3
SCALE = float(math.sqrt(float(D)))

_mesh = plsc.VectorSubcoreMesh(core_axis_name="c", subcore_axis_name="s")


@functools.partial(
    pl.kernel,
    out_type=jax.ShapeDtypeStruct((FLAT, D), jnp.float32),
    mesh=_mesh,
    scratch_types=[
        pltpu.VMEM((NCH, CHUNK), jnp.int32),
        pltpu.VMEM((CHUNK, D), jnp.float32),
        pltpu.VMEM((CHUNK, D), jnp.float32),
        pltpu.SemaphoreType.DMA,
        pltpu.SemaphoreType.DMA,
    ],
    compiler_params=pltpu.CompilerParams(use_tc_tiling_on_sc=True),
)
def _embed_gather(idx_hbm, table_hbm, out_hbm, idx_v, buf_a, buf_b, sem_a, sem_b):
    bufs = (buf_a, buf_b)
    sems = (sem_a, sem_b)
    wid = lax.axis_index("s") * NC + lax.axis_index("c")
    base = wid * PER_W

    pltpu.sync_copy(idx_hbm.at[pl.ds(wid * NCH, NCH)], idx_v)

    pltpu.async_copy(table_hbm.at[idx_v.at[0]], bufs[0], sems[0])

    for j in range(NCH):
        buf = bufs[j % 2]
        pltpu.make_async_copy(table_hbm.at[idx_v.at[j]], buf, sems[j % 2]).wait()
        if j + 1 < NCH:
            pltpu.async_copy(
                table_hbm.at[idx_v.at[j + 1]], bufs[(j + 1) % 2], sems[(j + 1) % 2]
            )

        def scale_row(r, _, buf=buf):
            for q in range(D // 16):
                buf[r, pl.ds(q * 16, 16)] = buf[r, pl.ds(q * 16, 16)] * SCALE
            return 0

        lax.fori_loop(0, CHUNK, scale_row, 0)
        pltpu.sync_copy(buf, out_hbm.at[pl.ds(base + j * CHUNK, CHUNK)])


def kernel(x, input_embedding_table):
    dummy = (jax.lax.broadcasted_iota(jnp.int32, (B, HPAD - H), 0) * (HPAD - H)
             + jax.lax.broadcasted_iota(jnp.int32, (B, HPAD - H), 1))
    idx = jnp.concatenate([x.astype(jnp.int32), dummy], axis=1)
    idx = idx.reshape(NW * NCH, CHUNK)
    framed = _embed_gather(idx, input_embedding_table)
    return framed.reshape(B, HPAD, D)[:, :H, :]
